# Initial kernel scaffold; baseline (speedup 1.0000x reference)
import jax
import jax.numpy as jnp
from jax.experimental import pallas as pl


def kernel(t, global_energy, features_0, energy_corrupted, layer, edge_index, graph_id, params):
    raise NotImplementedError("write your pallas kernel here")


# speedup vs baseline: 5.1624x; 5.1624x over previous
import functools
import math

import jax
import jax.numpy as jnp
from jax import lax
from jax.experimental import pallas as pl
from jax.experimental.pallas import tpu as pltpu
from jax.experimental.pallas import tpu_sc as plsc

_BLK = 2000


def _mmh(a, b):
    return lax.dot_general(a, b, (((1,), (0,)), ((), ())),
                           precision=lax.Precision.HIGHEST,
                           preferred_element_type=jnp.float32)


def _mmd(a, b):
    return lax.dot_general(a, b, (((1,), (0,)), ((), ())),
                           precision=lax.Precision.DEFAULT,
                           preferred_element_type=jnp.float32)


def _encode_body(feat8, layer2, gid2, t2, ge8,
                 le, Wc1a, Wc1b, bc1, Wc2, bc2,
                 Wp1, bp1, Wp2, bp2,
                 Wi1a, Wi1t, Wi1c, bi1, Wi2, bi2,
                 We1a, We1b, be1,
                 cell_o, h_o, A_o, B_o):
    f32 = jnp.float32
    t = t2[...]
    iot = lax.broadcasted_iota(jnp.int32, (1, 32), 1).astype(f32)
    freqs = jnp.exp(iot * (-math.log(200.0) / 32.0))
    x = t * freqs
    te = jnp.concatenate([jnp.cos(x), jnp.sin(x)], axis=1)
    cond = _mmd(jnp.maximum(_mmd(ge8[...], Wp1[...]) + bp1[...], 0.0), Wp2[...]) + bp2[...]
    G = _mmd(te, Wi1t[...]) + _mmd(cond, Wi1c[...])
    LT = _mmd(le[...], Wc1b[...])
    lay = layer2[...]
    ohl = (lax.broadcasted_iota(jnp.int32, (_BLK, 48), 1) == lay).astype(f32)
    c1 = _mmd(feat8[...], Wc1a[...]) + _mmh(ohl, LT) + bc1[...]
    cell = _mmd(jnp.maximum(c1, 0.0), Wc2[...]) + bc2[...]
    gid = gid2[...]
    ohg = (lax.broadcasted_iota(jnp.int32, (_BLK, 64), 1) == gid).astype(f32)
    h = _mmd(jnp.maximum(_mmd(cell, Wi1a[...]) + _mmh(ohg, G) + bi1[...], 0.0), Wi2[...]) + bi2[...]
    cell_o[...] = cell
    h_o[...] = h
    A = _mmd(h, We1a[...]) + be1[...]
    Bv = _mmd(h, We1b[...])
    A_o[0] = A[:, :32]
    A_o[1] = A[:, 32:]
    B_o[0] = Bv[:, :32]
    B_o[1] = Bv[:, 32:]


def _encode(feat8, layer2, gid2, t2, ge8, le, weights, N):
    grid = (N // _BLK,)
    row = lambda shape: pl.BlockSpec(shape, lambda i: (i, 0))
    full = lambda a: pl.BlockSpec(a.shape, lambda i: tuple([0] * a.ndim))
    in_specs = [row((_BLK, 8)), row((_BLK, 1)), row((_BLK, 1)),
                full(t2), full(ge8), full(le)] + [full(w) for w in weights]
    out_shape = [jax.ShapeDtypeStruct((N, 64), jnp.float32),
                 jax.ShapeDtypeStruct((N, 64), jnp.float32),
                 jax.ShapeDtypeStruct((2, N, 32), jnp.float32),
                 jax.ShapeDtypeStruct((2, N, 32), jnp.float32)]
    out_specs = [row((_BLK, 64)), row((_BLK, 64)),
                 pl.BlockSpec((2, _BLK, 32), lambda i: (0, i, 0)),
                 pl.BlockSpec((2, _BLK, 32), lambda i: (0, i, 0))]
    return pl.pallas_call(_encode_body, grid=grid, in_specs=in_specs,
                          out_specs=out_specs, out_shape=out_shape)(
        feat8, layer2, gid2, t2, ge8, le, *weights)


def _update_body(emit, h_i, S3, deg2,
                 We2a, We2b, be2, Wn1a, Wn1b, bn1, Wn2, bn2, *rest):
    if emit:
        We1a, We1b, be1, h_o, A_o, B_o = rest
    else:
        (h_o,) = rest
    h = h_i[...]
    agg = _mmh(S3[0], We2a[...]) + _mmh(S3[1], We2b[...]) + deg2[...] * be2[...]
    hn = h + _mmd(jnp.maximum(_mmd(h, Wn1a[...]) + _mmd(agg, Wn1b[...]) + bn1[...], 0.0), Wn2[...]) + bn2[...]
    h_o[...] = hn
    if emit:
        A = _mmd(hn, We1a[...]) + be1[...]
        Bv = _mmd(hn, We1b[...])
        A_o[0] = A[:, :32]
        A_o[1] = A[:, 32:]
        B_o[0] = Bv[:, :32]
        B_o[1] = Bv[:, 32:]


def _update(h, S3, deg2, weights, emit, N):
    grid = (N // _BLK,)
    row = lambda shape: pl.BlockSpec(shape, lambda i: (i, 0))
    full = lambda a: pl.BlockSpec(a.shape, lambda i: tuple([0] * a.ndim))
    in_specs = [row((_BLK, 64)), pl.BlockSpec((2, _BLK, 32), lambda i: (0, i, 0)),
                row((_BLK, 1))] + [full(w) for w in weights]
    out_shape = [jax.ShapeDtypeStruct((N, 64), jnp.float32)]
    out_specs = [row((_BLK, 64))]
    if emit:
        out_shape += [jax.ShapeDtypeStruct((2, N, 32), jnp.float32)] * 2
        out_specs += [pl.BlockSpec((2, _BLK, 32), lambda i: (0, i, 0))] * 2
    return pl.pallas_call(functools.partial(_update_body, emit), grid=grid,
                          in_specs=in_specs, out_specs=out_specs,
                          out_shape=out_shape)(h, S3, deg2, *weights)


def _final_body(h_i, cell_i, gid2, t2, ge8,
                Wp1, bp1, Wp2, bp2,
                Wz1h, Wz1cell, Wz1t, Wz1c, bz1, Wz2, bz2, F_o):
    f32 = jnp.float32
    t = t2[...]
    iot = lax.broadcasted_iota(jnp.int32, (1, 32), 1).astype(f32)
    freqs = jnp.exp(iot * (-math.log(200.0) / 32.0))
    x = t * freqs
    te = jnp.concatenate([jnp.cos(x), jnp.sin(x)], axis=1)
    cond = _mmd(jnp.maximum(_mmd(ge8[...], Wp1[...]) + bp1[...], 0.0), Wp2[...]) + bp2[...]
    Gn = _mmd(te, Wz1t[...]) + _mmd(cond, Wz1c[...])
    gid = gid2[...]
    ohg = (lax.broadcasted_iota(jnp.int32, (_BLK, 64), 1) == gid).astype(f32)
    u1 = _mmd(h_i[...], Wz1h[...]) + _mmd(cell_i[...], Wz1cell[...]) + _mmh(ohg, Gn) + bz1[...]
    F_o[...] = _mmd(jnp.maximum(u1, 0.0), Wz2[...]) + bz2[...]


def _final(h, cell, gid2, t2, ge8, weights, N):
    grid = (N // _BLK,)
    row = lambda shape: pl.BlockSpec(shape, lambda i: (i, 0))
    full = lambda a: pl.BlockSpec(a.shape, lambda i: tuple([0] * a.ndim))
    in_specs = [row((_BLK, 64)), row((_BLK, 64)), row((_BLK, 1)),
                full(t2), full(ge8)] + [full(w) for w in weights]
    return pl.pallas_call(_final_body, grid=grid, in_specs=in_specs,
                          out_specs=row((_BLK, 1)),
                          out_shape=jax.ShapeDtypeStruct((N, 1), jnp.float32))(
        h, cell, gid2, t2, ge8, *weights)


def _make_edge_kernel(N, E, compute_deg):
    NS = 16
    EperS = E // NS
    C = 400
    NCH = EperS // C
    RperS = -(-(N // NS) // 8) * 8
    NP = NS * RperS
    assert EperS % C == 0

    mesh = plsc.VectorSubcoreMesh(core_axis_name="c", subcore_axis_name="s")

    out_type = [jax.ShapeDtypeStruct((2 * NP, 32), jnp.float32)]
    if compute_deg:
        out_type.append(jax.ShapeDtypeStruct((N,), jnp.float32))

    scratch_types = [
        pltpu.VMEM((C,), jnp.int32),
        pltpu.VMEM((C,), jnp.int32),
        pltpu.VMEM((C,), jnp.int32),
        pltpu.VMEM((C, 32), jnp.float32),
        pltpu.VMEM((C, 32), jnp.float32),
        pltpu.VMEM((C,), jnp.float32),
        pltpu.VMEM_SHARED((NP, 32), jnp.float32),
    ]
    if compute_deg:
        scratch_types.append(pltpu.VMEM_SHARED((N,), jnp.float32))
    scratch_types += [pltpu.SemaphoreType.DMA, pltpu.SemaphoreType.DMA]

    def body(A_h, B_h, src_h, dst_h, *refs):
        if compute_deg:
            (S_out, deg_out, src_v, dst_v, dstg_v, bufA, bufB, ones_v,
             S_sh, deg_sh, semA, semB) = refs
        else:
            (S_out, src_v, dst_v, dstg_v, bufA, bufB, ones_v,
             S_sh, semA, semB) = refs
        c = lax.axis_index("c")
        s = lax.axis_index("s")
        cN = c * N
        cNP = c * NP
        zeros16 = jnp.zeros((16,), jnp.float32)

        def zrow(i, _):
            bufA[i, pl.ds(0, 16)] = zeros16
            bufA[i, pl.ds(16, 16)] = zeros16
            return _
        lax.fori_loop(0, C, zrow, None)

        def zv(i, _):
            ones_v[pl.ds(i * 16, 16)] = zeros16
            return _
        lax.fori_loop(0, C // 16, zv, None)

        r0 = s * RperS
        n_full = RperS // C
        rem = RperS - n_full * C
        for j in range(n_full):
            pltpu.sync_copy(bufA, S_sh.at[pl.ds(r0 + j * C, C)])
        if rem:
            pltpu.sync_copy(bufA.at[pl.ds(0, rem)], S_sh.at[pl.ds(r0 + n_full * C, rem)])

        if compute_deg:
            DEG_PER = N // 10
            dfull, drem = DEG_PER // C, DEG_PER % C

            @pl.when(jnp.logical_and(c == 0, s < 10))
            def _():
                for j in range(dfull):
                    pltpu.sync_copy(ones_v, deg_sh.at[pl.ds(s * DEG_PER + j * C, C)])
                if drem:
                    pltpu.sync_copy(ones_v.at[pl.ds(0, drem)],
                                    deg_sh.at[pl.ds(s * DEG_PER + dfull * C, drem)])

            ones16 = jnp.full((16,), 1.0, jnp.float32)

            def ov(i, _):
                ones_v[pl.ds(i * 16, 16)] = ones16
                return _
            lax.fori_loop(0, C // 16, ov, None)

        plsc.subcore_barrier()

        ebase = s * EperS

        def chunk(k, _):
            b0 = ebase + k * C
            pltpu.sync_copy(src_h.at[pl.ds(b0, C)], src_v)
            pltpu.sync_copy(dst_h.at[pl.ds(b0, C)], dst_v)

            def adj(i, _):
                src_v[pl.ds(i * 16, 16)] = src_v[pl.ds(i * 16, 16)] + cN
                dstg_v[pl.ds(i * 16, 16)] = dst_v[pl.ds(i * 16, 16)] + cN
                return _
            lax.fori_loop(0, C // 16, adj, None)

            cpA = pltpu.async_copy(A_h.at[src_v], bufA, semA)
            cpB = pltpu.async_copy(B_h.at[dstg_v], bufB, semB)
            cpA.wait()
            cpB.wait()

            def bf16_round(v):
                u = lax.bitcast_convert_type(v, jnp.int32)
                bit = jnp.bitwise_and(lax.shift_right_logical(u, 16), 1)
                r = jnp.bitwise_and(u + 32767 + bit, jnp.int32(-65536))
                return lax.bitcast_convert_type(r, jnp.float32)

            def row(i, _):
                a0 = bufA[i, pl.ds(0, 16)]
                b0_ = bufB[i, pl.ds(0, 16)]
                bufA[i, pl.ds(0, 16)] = bf16_round(jnp.maximum(a0 + b0_, 0.0))
                a1 = bufA[i, pl.ds(16, 16)]
                b1_ = bufB[i, pl.ds(16, 16)]
                bufA[i, pl.ds(16, 16)] = bf16_round(jnp.maximum(a1 + b1_, 0.0))
                return _
            lax.fori_loop(0, C, row, None)

            pltpu.sync_copy(bufA, S_sh.at[dst_v], add=True)
            if compute_deg:
                @pl.when(c == 0)
                def _():
                    pltpu.sync_copy(ones_v, deg_sh.at[dst_v], add=True)
            return _
        lax.fori_loop(0, NCH, chunk, None)

        plsc.subcore_barrier()

        o0 = cNP + r0
        for j in range(n_full):
            pltpu.sync_copy(S_sh.at[pl.ds(r0 + j * C, C)], bufA)
            pltpu.sync_copy(bufA, S_out.at[pl.ds(o0 + j * C, C)])
        if rem:
            pltpu.sync_copy(S_sh.at[pl.ds(r0 + n_full * C, rem)], bufA.at[pl.ds(0, rem)])
            pltpu.sync_copy(bufA.at[pl.ds(0, rem)], S_out.at[pl.ds(o0 + n_full * C, rem)])

        if compute_deg:
            DEG_PER2 = N // 10
            dfull2, drem2 = DEG_PER2 // C, DEG_PER2 % C

            @pl.when(jnp.logical_and(c == 0, s < 10))
            def _():
                for j in range(dfull2):
                    pltpu.sync_copy(deg_sh.at[pl.ds(s * DEG_PER2 + j * C, C)], ones_v)
                    pltpu.sync_copy(ones_v, deg_out.at[pl.ds(s * DEG_PER2 + j * C, C)])
                if drem2:
                    pltpu.sync_copy(deg_sh.at[pl.ds(s * DEG_PER2 + dfull2 * C, drem2)],
                                    ones_v.at[pl.ds(0, drem2)])
                    pltpu.sync_copy(ones_v.at[pl.ds(0, drem2)],
                                    deg_out.at[pl.ds(s * DEG_PER2 + dfull2 * C, drem2)])

    return pl.kernel(body, mesh=mesh, out_type=out_type,
                     scratch_types=scratch_types,
                     compiler_params=pltpu.CompilerParams(
                         use_tc_tiling_on_sc=False))


def kernel(t, global_energy, features_0, energy_corrupted, layer, edge_index,
           graph_id, params):
    N = features_0.shape[0]
    E = edge_index.shape[1]
    f32 = jnp.float32

    (Wp1, bp1), (Wp2, bp2) = params['particle']
    (Wc1, bc1), (Wc2, bc2) = params['cell']
    (Wi1, bi1), (Wi2, bi2) = params['init']
    (Wz1, bz1), (Wz2, bz2) = params['noise']

    Wp1_8 = jnp.zeros((8, 64), f32).at[:4].set(Wp1)
    Wc1a = jnp.zeros((8, 64), f32).at[:5].set(Wc1[:5])
    Wc1b = Wc1[5:21]
    le48 = jnp.zeros((48, 16), f32).at[:45].set(params['layer_emb'])
    row2 = lambda b: b.reshape(1, -1)

    feat8 = jnp.concatenate(
        [energy_corrupted, features_0, jnp.zeros((N, 3), f32)], axis=1)
    layer2 = layer.reshape(N, 1).astype(jnp.int32)
    gid2 = graph_id.reshape(N, 1).astype(jnp.int32)
    t2 = t.reshape(-1, 1)
    ge8 = jnp.zeros((global_energy.shape[0], 8), f32).at[:, :4].set(global_energy)

    (We1_0, be1_0), (We2_0, be2_0) = params['mpnn'][0][0]
    (Wn1_0, bn1_0), (Wn2_0, bn2_0) = params['mpnn'][0][1]
    (We1_1, be1_1), (We2_1, be2_1) = params['mpnn'][1][0]
    (Wn1_1, bn1_1), (Wn2_1, bn2_1) = params['mpnn'][1][1]

    enc_w = [Wc1a, Wc1b, row2(bc1), Wc2, row2(bc2),
             Wp1_8, row2(bp1), Wp2, row2(bp2),
             Wi1[:64], Wi1[64:128], Wi1[128:192], row2(bi1), Wi2, row2(bi2),
             We1_0[:64], We1_0[64:], row2(be1_0)]
    cell, h, A3, B3 = _encode(feat8, layer2, gid2, t2, ge8, le48, enc_w, N)

    src = edge_index[0].astype(jnp.int32)
    dst = edge_index[1].astype(jnp.int32)

    NP = 16 * (-(-(N // 16) // 8) * 8)

    edge_k1 = _make_edge_kernel(N, E, compute_deg=True)
    S_flat, deg = edge_k1(A3.reshape(2 * N, 32), B3.reshape(2 * N, 32), src, dst)
    deg2 = deg.reshape(N, 1)

    bf = lambda w: w.astype(jnp.bfloat16).astype(f32)
    upd_w0 = [bf(We2_0[:32]), bf(We2_0[32:]), row2(be2_0),
              Wn1_0[:64], Wn1_0[64:], row2(bn1_0), Wn2_0, row2(bn2_0),
              We1_1[:64], We1_1[64:], row2(be1_1)]
    h, A3, B3 = _update(h, S_flat.reshape(2, NP, 32)[:, :N], deg2, upd_w0, True, N)

    edge_k2 = _make_edge_kernel(N, E, compute_deg=False)
    S_flat = edge_k2(A3.reshape(2 * N, 32), B3.reshape(2 * N, 32), src, dst)
    if isinstance(S_flat, (list, tuple)):
        S_flat = S_flat[0]

    upd_w1 = [bf(We2_1[:32]), bf(We2_1[32:]), row2(be2_1),
              Wn1_1[:64], Wn1_1[64:], row2(bn1_1), Wn2_1, row2(bn2_1)]
    (h,) = _update(h, S_flat.reshape(2, NP, 32)[:, :N], deg2, upd_w1, False, N)

    fin_w = [Wp1_8, row2(bp1), Wp2, row2(bp2),
             Wz1[:64], Wz1[64:128], Wz1[128:192], Wz1[192:256],
             row2(bz1), Wz2, row2(bz2)]
    return _final(h, cell, gid2, t2, ge8, fin_w, N)


# speedup vs baseline: 6.6093x; 1.2803x over previous
import functools
import math

import jax
import jax.numpy as jnp
from jax import lax
from jax.experimental import pallas as pl
from jax.experimental.pallas import tpu as pltpu
from jax.experimental.pallas import tpu_sc as plsc

_BLK = 2000


def _mmh(a, b):
    return lax.dot_general(a, b, (((1,), (0,)), ((), ())),
                           precision=lax.Precision.HIGHEST,
                           preferred_element_type=jnp.float32)


def _mmd(a, b):
    return lax.dot_general(a, b, (((1,), (0,)), ((), ())),
                           precision=lax.Precision.DEFAULT,
                           preferred_element_type=jnp.float32)


def _encode_body(feat8, layer2, gid2, t2, ge8,
                 le, Wc1a, Wc1b, bc1, Wc2, bc2,
                 Wp1, bp1, Wp2, bp2,
                 Wi1a, Wi1t, Wi1c, bi1, Wi2, bi2,
                 We1a, We1b, be1,
                 cell_o, h_o, A_o, B_o):
    f32 = jnp.float32
    t = t2[...]
    iot = lax.broadcasted_iota(jnp.int32, (1, 32), 1).astype(f32)
    freqs = jnp.exp(iot * (-math.log(200.0) / 32.0))
    x = t * freqs
    te = jnp.concatenate([jnp.cos(x), jnp.sin(x)], axis=1)
    cond = _mmd(jnp.maximum(_mmd(ge8[...], Wp1[...]) + bp1[...], 0.0), Wp2[...]) + bp2[...]
    G = _mmd(te, Wi1t[...]) + _mmd(cond, Wi1c[...])
    LT = _mmd(le[...], Wc1b[...])
    lay = layer2[...]
    ohl = (lax.broadcasted_iota(jnp.int32, (_BLK, 48), 1) == lay).astype(f32)
    c1 = _mmd(feat8[...], Wc1a[...]) + _mmh(ohl, LT) + bc1[...]
    cell = _mmd(jnp.maximum(c1, 0.0), Wc2[...]) + bc2[...]
    gid = gid2[...]
    ohg = (lax.broadcasted_iota(jnp.int32, (_BLK, 64), 1) == gid).astype(f32)
    h = _mmd(jnp.maximum(_mmd(cell, Wi1a[...]) + _mmh(ohg, G) + bi1[...], 0.0), Wi2[...]) + bi2[...]
    cell_o[...] = cell
    h_o[...] = h
    A = _mmd(h, We1a[...]) + be1[...]
    Bv = _mmd(h, We1b[...])
    A_o[0] = A[:, :32]
    A_o[1] = A[:, 32:]
    B_o[0] = Bv[:, :32]
    B_o[1] = Bv[:, 32:]


def _encode(feat8, layer2, gid2, t2, ge8, le, weights, N):
    grid = (N // _BLK,)
    row = lambda shape: pl.BlockSpec(shape, lambda i: (i, 0))
    full = lambda a: pl.BlockSpec(a.shape, lambda i: tuple([0] * a.ndim))
    in_specs = [row((_BLK, 8)), row((_BLK, 1)), row((_BLK, 1)),
                full(t2), full(ge8), full(le)] + [full(w) for w in weights]
    out_shape = [jax.ShapeDtypeStruct((N, 64), jnp.float32),
                 jax.ShapeDtypeStruct((N, 64), jnp.float32),
                 jax.ShapeDtypeStruct((2, N, 32), jnp.float32),
                 jax.ShapeDtypeStruct((2, N, 32), jnp.float32)]
    out_specs = [row((_BLK, 64)), row((_BLK, 64)),
                 pl.BlockSpec((2, _BLK, 32), lambda i: (0, i, 0)),
                 pl.BlockSpec((2, _BLK, 32), lambda i: (0, i, 0))]
    return pl.pallas_call(_encode_body, grid=grid, in_specs=in_specs,
                          out_specs=out_specs, out_shape=out_shape)(
        feat8, layer2, gid2, t2, ge8, le, *weights)


def _update_body(emit, h_i, S3, deg2,
                 We2a, We2b, be2, Wn1a, Wn1b, bn1, Wn2, bn2, *rest):
    if emit:
        We1a, We1b, be1, h_o, A_o, B_o = rest
    else:
        (h_o,) = rest
    h = h_i[...]
    agg = _mmh(S3[0], We2a[...]) + _mmh(S3[1], We2b[...]) + deg2[...] * be2[...]
    hn = h + _mmd(jnp.maximum(_mmd(h, Wn1a[...]) + _mmd(agg, Wn1b[...]) + bn1[...], 0.0), Wn2[...]) + bn2[...]
    h_o[...] = hn
    if emit:
        A = _mmd(hn, We1a[...]) + be1[...]
        Bv = _mmd(hn, We1b[...])
        A_o[0] = A[:, :32]
        A_o[1] = A[:, 32:]
        B_o[0] = Bv[:, :32]
        B_o[1] = Bv[:, 32:]


def _update(h, S3, deg2, weights, emit, N):
    grid = (N // _BLK,)
    row = lambda shape: pl.BlockSpec(shape, lambda i: (i, 0))
    full = lambda a: pl.BlockSpec(a.shape, lambda i: tuple([0] * a.ndim))
    in_specs = [row((_BLK, 64)), pl.BlockSpec((2, _BLK, 32), lambda i: (0, i, 0)),
                row((_BLK, 1))] + [full(w) for w in weights]
    out_shape = [jax.ShapeDtypeStruct((N, 64), jnp.float32)]
    out_specs = [row((_BLK, 64))]
    if emit:
        out_shape += [jax.ShapeDtypeStruct((2, N, 32), jnp.float32)] * 2
        out_specs += [pl.BlockSpec((2, _BLK, 32), lambda i: (0, i, 0))] * 2
    return pl.pallas_call(functools.partial(_update_body, emit), grid=grid,
                          in_specs=in_specs, out_specs=out_specs,
                          out_shape=out_shape)(h, S3, deg2, *weights)


def _final_body(h_i, cell_i, gid2, t2, ge8,
                Wp1, bp1, Wp2, bp2,
                Wz1h, Wz1cell, Wz1t, Wz1c, bz1, Wz2, bz2, F_o):
    f32 = jnp.float32
    t = t2[...]
    iot = lax.broadcasted_iota(jnp.int32, (1, 32), 1).astype(f32)
    freqs = jnp.exp(iot * (-math.log(200.0) / 32.0))
    x = t * freqs
    te = jnp.concatenate([jnp.cos(x), jnp.sin(x)], axis=1)
    cond = _mmd(jnp.maximum(_mmd(ge8[...], Wp1[...]) + bp1[...], 0.0), Wp2[...]) + bp2[...]
    Gn = _mmd(te, Wz1t[...]) + _mmd(cond, Wz1c[...])
    gid = gid2[...]
    ohg = (lax.broadcasted_iota(jnp.int32, (_BLK, 64), 1) == gid).astype(f32)
    u1 = _mmd(h_i[...], Wz1h[...]) + _mmd(cell_i[...], Wz1cell[...]) + _mmh(ohg, Gn) + bz1[...]
    F_o[...] = _mmd(jnp.maximum(u1, 0.0), Wz2[...]) + bz2[...]


def _final(h, cell, gid2, t2, ge8, weights, N):
    grid = (N // _BLK,)
    row = lambda shape: pl.BlockSpec(shape, lambda i: (i, 0))
    full = lambda a: pl.BlockSpec(a.shape, lambda i: tuple([0] * a.ndim))
    in_specs = [row((_BLK, 64)), row((_BLK, 64)), row((_BLK, 1)),
                full(t2), full(ge8)] + [full(w) for w in weights]
    return pl.pallas_call(_final_body, grid=grid, in_specs=in_specs,
                          out_specs=row((_BLK, 1)),
                          out_shape=jax.ShapeDtypeStruct((N, 1), jnp.float32))(
        h, cell, gid2, t2, ge8, *weights)


def _make_edge_kernel(N, E, compute_deg):
    NS = 16
    EperS = E // NS
    C = 400
    NCH = EperS // C
    RperS = -(-(N // NS) // 8) * 8
    NP = NS * RperS
    assert EperS % C == 0

    mesh = plsc.VectorSubcoreMesh(core_axis_name="c", subcore_axis_name="s")

    out_type = [jax.ShapeDtypeStruct((2 * NP, 32), jnp.float32)]
    if compute_deg:
        out_type.append(jax.ShapeDtypeStruct((N,), jnp.float32))

    scratch_types = [
        pltpu.VMEM((C,), jnp.int32),
        pltpu.VMEM((C,), jnp.int32),
        pltpu.VMEM((C,), jnp.int32),
        pltpu.VMEM((C, 32), jnp.float32),
        pltpu.VMEM((C, 32), jnp.float32),
        pltpu.VMEM((C,), jnp.float32),
        pltpu.VMEM_SHARED((NP, 32), jnp.float32),
    ]
    if compute_deg:
        scratch_types.append(pltpu.VMEM_SHARED((N,), jnp.float32))
    scratch_types += [pltpu.SemaphoreType.DMA, pltpu.SemaphoreType.DMA]

    def body(A_h, B_h, src_h, dst_h, *refs):
        if compute_deg:
            (S_out, deg_out, src_v, dst_v, dstg_v, bufA, bufB, ones_v,
             S_sh, deg_sh, semA, semB) = refs
        else:
            (S_out, src_v, dst_v, dstg_v, bufA, bufB, ones_v,
             S_sh, semA, semB) = refs
        c = lax.axis_index("c")
        s = lax.axis_index("s")
        cN = c * N
        cNP = c * NP
        zeros16 = jnp.zeros((16,), jnp.float32)

        def zrow(i, _):
            bufA[i, pl.ds(0, 16)] = zeros16
            bufA[i, pl.ds(16, 16)] = zeros16
            return _
        lax.fori_loop(0, C, zrow, None)

        def zv(i, _):
            ones_v[pl.ds(i * 16, 16)] = zeros16
            return _
        lax.fori_loop(0, C // 16, zv, None)

        r0 = s * RperS
        n_full = RperS // C
        rem = RperS - n_full * C
        for j in range(n_full):
            pltpu.sync_copy(bufA, S_sh.at[pl.ds(r0 + j * C, C)])
        if rem:
            pltpu.sync_copy(bufA.at[pl.ds(0, rem)], S_sh.at[pl.ds(r0 + n_full * C, rem)])

        if compute_deg:
            DEG_PER = N // 10
            dfull, drem = DEG_PER // C, DEG_PER % C

            @pl.when(jnp.logical_and(c == 0, s < 10))
            def _():
                for j in range(dfull):
                    pltpu.sync_copy(ones_v, deg_sh.at[pl.ds(s * DEG_PER + j * C, C)])
                if drem:
                    pltpu.sync_copy(ones_v.at[pl.ds(0, drem)],
                                    deg_sh.at[pl.ds(s * DEG_PER + dfull * C, drem)])

            ones16 = jnp.full((16,), 1.0, jnp.float32)

            def ov(i, _):
                ones_v[pl.ds(i * 16, 16)] = ones16
                return _
            lax.fori_loop(0, C // 16, ov, None)

        plsc.subcore_barrier()

        ebase = s * EperS

        def chunk(k, _):
            b0 = ebase + k * C
            pltpu.sync_copy(src_h.at[pl.ds(b0, C)], src_v)
            pltpu.sync_copy(dst_h.at[pl.ds(b0, C)], dst_v)

            @plsc.parallel_loop(0, C // 16, 1, unroll=4)
            def _(i):
                src_v[pl.ds(i * 16, 16)] = src_v[pl.ds(i * 16, 16)] + cN
                dstg_v[pl.ds(i * 16, 16)] = dst_v[pl.ds(i * 16, 16)] + cN

            cpA = pltpu.async_copy(A_h.at[src_v], bufA, semA)
            cpB = pltpu.async_copy(B_h.at[dstg_v], bufB, semB)
            cpA.wait()
            cpB.wait()

            def bf16_round(v):
                u = lax.bitcast_convert_type(v, jnp.int32)
                bit = jnp.bitwise_and(lax.shift_right_logical(u, 16), 1)
                r = jnp.bitwise_and(u + 32767 + bit, jnp.int32(-65536))
                return lax.bitcast_convert_type(r, jnp.float32)

            @plsc.parallel_loop(0, C, 1, unroll=8)
            def _(i):
                a0 = bufA[i, pl.ds(0, 16)]
                b0_ = bufB[i, pl.ds(0, 16)]
                bufA[i, pl.ds(0, 16)] = bf16_round(jnp.maximum(a0 + b0_, 0.0))
                a1 = bufA[i, pl.ds(16, 16)]
                b1_ = bufB[i, pl.ds(16, 16)]
                bufA[i, pl.ds(16, 16)] = bf16_round(jnp.maximum(a1 + b1_, 0.0))

            pltpu.sync_copy(bufA, S_sh.at[dst_v], add=True)
            if compute_deg:
                @pl.when(c == 0)
                def _():
                    pltpu.sync_copy(ones_v, deg_sh.at[dst_v], add=True)
            return _
        lax.fori_loop(0, NCH, chunk, None)

        plsc.subcore_barrier()

        o0 = cNP + r0
        for j in range(n_full):
            pltpu.sync_copy(S_sh.at[pl.ds(r0 + j * C, C)], bufA)
            pltpu.sync_copy(bufA, S_out.at[pl.ds(o0 + j * C, C)])
        if rem:
            pltpu.sync_copy(S_sh.at[pl.ds(r0 + n_full * C, rem)], bufA.at[pl.ds(0, rem)])
            pltpu.sync_copy(bufA.at[pl.ds(0, rem)], S_out.at[pl.ds(o0 + n_full * C, rem)])

        if compute_deg:
            DEG_PER2 = N // 10
            dfull2, drem2 = DEG_PER2 // C, DEG_PER2 % C

            @pl.when(jnp.logical_and(c == 0, s < 10))
            def _():
                for j in range(dfull2):
                    pltpu.sync_copy(deg_sh.at[pl.ds(s * DEG_PER2 + j * C, C)], ones_v)
                    pltpu.sync_copy(ones_v, deg_out.at[pl.ds(s * DEG_PER2 + j * C, C)])
                if drem2:
                    pltpu.sync_copy(deg_sh.at[pl.ds(s * DEG_PER2 + dfull2 * C, drem2)],
                                    ones_v.at[pl.ds(0, drem2)])
                    pltpu.sync_copy(ones_v.at[pl.ds(0, drem2)],
                                    deg_out.at[pl.ds(s * DEG_PER2 + dfull2 * C, drem2)])

    return pl.kernel(body, mesh=mesh, out_type=out_type,
                     scratch_types=scratch_types,
                     compiler_params=pltpu.CompilerParams(
                         use_tc_tiling_on_sc=False))


def kernel(t, global_energy, features_0, energy_corrupted, layer, edge_index,
           graph_id, params):
    N = features_0.shape[0]
    E = edge_index.shape[1]
    f32 = jnp.float32

    (Wp1, bp1), (Wp2, bp2) = params['particle']
    (Wc1, bc1), (Wc2, bc2) = params['cell']
    (Wi1, bi1), (Wi2, bi2) = params['init']
    (Wz1, bz1), (Wz2, bz2) = params['noise']

    Wp1_8 = jnp.zeros((8, 64), f32).at[:4].set(Wp1)
    Wc1a = jnp.zeros((8, 64), f32).at[:5].set(Wc1[:5])
    Wc1b = Wc1[5:21]
    le48 = jnp.zeros((48, 16), f32).at[:45].set(params['layer_emb'])
    row2 = lambda b: b.reshape(1, -1)

    feat8 = jnp.concatenate(
        [energy_corrupted, features_0, jnp.zeros((N, 3), f32)], axis=1)
    layer2 = layer.reshape(N, 1).astype(jnp.int32)
    gid2 = graph_id.reshape(N, 1).astype(jnp.int32)
    t2 = t.reshape(-1, 1)
    ge8 = jnp.zeros((global_energy.shape[0], 8), f32).at[:, :4].set(global_energy)

    (We1_0, be1_0), (We2_0, be2_0) = params['mpnn'][0][0]
    (Wn1_0, bn1_0), (Wn2_0, bn2_0) = params['mpnn'][0][1]
    (We1_1, be1_1), (We2_1, be2_1) = params['mpnn'][1][0]
    (Wn1_1, bn1_1), (Wn2_1, bn2_1) = params['mpnn'][1][1]

    enc_w = [Wc1a, Wc1b, row2(bc1), Wc2, row2(bc2),
             Wp1_8, row2(bp1), Wp2, row2(bp2),
             Wi1[:64], Wi1[64:128], Wi1[128:192], row2(bi1), Wi2, row2(bi2),
             We1_0[:64], We1_0[64:], row2(be1_0)]
    cell, h, A3, B3 = _encode(feat8, layer2, gid2, t2, ge8, le48, enc_w, N)

    src = edge_index[0].astype(jnp.int32)
    dst = edge_index[1].astype(jnp.int32)

    NP = 16 * (-(-(N // 16) // 8) * 8)

    edge_k1 = _make_edge_kernel(N, E, compute_deg=True)
    S_flat, deg = edge_k1(A3.reshape(2 * N, 32), B3.reshape(2 * N, 32), src, dst)
    deg2 = deg.reshape(N, 1)

    bf = lambda w: w.astype(jnp.bfloat16).astype(f32)
    upd_w0 = [bf(We2_0[:32]), bf(We2_0[32:]), row2(be2_0),
              Wn1_0[:64], Wn1_0[64:], row2(bn1_0), Wn2_0, row2(bn2_0),
              We1_1[:64], We1_1[64:], row2(be1_1)]
    h, A3, B3 = _update(h, S_flat.reshape(2, NP, 32)[:, :N], deg2, upd_w0, True, N)

    edge_k2 = _make_edge_kernel(N, E, compute_deg=False)
    S_flat = edge_k2(A3.reshape(2 * N, 32), B3.reshape(2 * N, 32), src, dst)
    if isinstance(S_flat, (list, tuple)):
        S_flat = S_flat[0]

    upd_w1 = [bf(We2_1[:32]), bf(We2_1[32:]), row2(be2_1),
              Wn1_1[:64], Wn1_1[64:], row2(bn1_1), Wn2_1, row2(bn2_1)]
    (h,) = _update(h, S_flat.reshape(2, NP, 32)[:, :N], deg2, upd_w1, False, N)

    fin_w = [Wp1_8, row2(bp1), Wp2, row2(bp2),
             Wz1[:64], Wz1[64:128], Wz1[128:192], Wz1[192:256],
             row2(bz1), Wz2, row2(bz2)]
    return _final(h, cell, gid2, t2, ge8, fin_w, N)


# speedup vs baseline: 6.6923x; 1.0126x over previous
import functools
import math

import jax
import jax.numpy as jnp
from jax import lax
from jax.experimental import pallas as pl
from jax.experimental.pallas import tpu as pltpu
from jax.experimental.pallas import tpu_sc as plsc

_BLK = 2000


def _mmh(a, b):
    return lax.dot_general(a, b, (((1,), (0,)), ((), ())),
                           precision=lax.Precision.HIGHEST,
                           preferred_element_type=jnp.float32)


def _mmd(a, b):
    return lax.dot_general(a, b, (((1,), (0,)), ((), ())),
                           precision=lax.Precision.DEFAULT,
                           preferred_element_type=jnp.float32)


def _encode_body(feat8, layer2, gid2, t2, ge8,
                 le, Wc1a, Wc1b, bc1, Wc2, bc2,
                 Wp1, bp1, Wp2, bp2,
                 Wi1a, Wi1t, Wi1c, bi1, Wi2, bi2,
                 We1a, We1b, be1,
                 cell_o, h_o, A_o, B_o):
    f32 = jnp.float32
    t = t2[...]
    iot = lax.broadcasted_iota(jnp.int32, (1, 32), 1).astype(f32)
    freqs = jnp.exp(iot * (-math.log(200.0) / 32.0))
    x = t * freqs
    te = jnp.concatenate([jnp.cos(x), jnp.sin(x)], axis=1)
    cond = _mmd(jnp.maximum(_mmd(ge8[...], Wp1[...]) + bp1[...], 0.0), Wp2[...]) + bp2[...]
    G = _mmd(te, Wi1t[...]) + _mmd(cond, Wi1c[...])
    LT = _mmd(le[...], Wc1b[...])
    lay = layer2[...]
    ohl = (lax.broadcasted_iota(jnp.int32, (_BLK, 48), 1) == lay).astype(f32)
    c1 = _mmd(feat8[...], Wc1a[...]) + _mmh(ohl, LT) + bc1[...]
    cell = _mmd(jnp.maximum(c1, 0.0), Wc2[...]) + bc2[...]
    gid = gid2[...]
    ohg = (lax.broadcasted_iota(jnp.int32, (_BLK, 64), 1) == gid).astype(f32)
    h = _mmd(jnp.maximum(_mmd(cell, Wi1a[...]) + _mmh(ohg, G) + bi1[...], 0.0), Wi2[...]) + bi2[...]
    cell_o[...] = cell
    h_o[...] = h
    A = _mmd(h, We1a[...]) + be1[...]
    Bv = _mmd(h, We1b[...])
    A_o[0] = A[:, :32]
    A_o[1] = A[:, 32:]
    B_o[0] = Bv[:, :32]
    B_o[1] = Bv[:, 32:]


def _encode(feat8, layer2, gid2, t2, ge8, le, weights, N):
    grid = (N // _BLK,)
    row = lambda shape: pl.BlockSpec(shape, lambda i: (i, 0))
    full = lambda a: pl.BlockSpec(a.shape, lambda i: tuple([0] * a.ndim))
    in_specs = [row((_BLK, 8)), row((_BLK, 1)), row((_BLK, 1)),
                full(t2), full(ge8), full(le)] + [full(w) for w in weights]
    out_shape = [jax.ShapeDtypeStruct((N, 64), jnp.float32),
                 jax.ShapeDtypeStruct((N, 64), jnp.float32),
                 jax.ShapeDtypeStruct((2, N, 32), jnp.float32),
                 jax.ShapeDtypeStruct((2, N, 32), jnp.float32)]
    out_specs = [row((_BLK, 64)), row((_BLK, 64)),
                 pl.BlockSpec((2, _BLK, 32), lambda i: (0, i, 0)),
                 pl.BlockSpec((2, _BLK, 32), lambda i: (0, i, 0))]
    return pl.pallas_call(_encode_body, grid=grid, in_specs=in_specs,
                          out_specs=out_specs, out_shape=out_shape)(
        feat8, layer2, gid2, t2, ge8, le, *weights)


def _update_body(emit, h_i, S3, deg2,
                 We2a, We2b, be2, Wn1a, Wn1b, bn1, Wn2, bn2, *rest):
    if emit:
        We1a, We1b, be1, h_o, A_o, B_o = rest
    else:
        (h_o,) = rest
    h = h_i[...]
    agg = _mmh(S3[0], We2a[...]) + _mmh(S3[1], We2b[...]) + deg2[...] * be2[...]
    hn = h + _mmd(jnp.maximum(_mmd(h, Wn1a[...]) + _mmd(agg, Wn1b[...]) + bn1[...], 0.0), Wn2[...]) + bn2[...]
    h_o[...] = hn
    if emit:
        A = _mmd(hn, We1a[...]) + be1[...]
        Bv = _mmd(hn, We1b[...])
        A_o[0] = A[:, :32]
        A_o[1] = A[:, 32:]
        B_o[0] = Bv[:, :32]
        B_o[1] = Bv[:, 32:]


def _update(h, S3, deg2, weights, emit, N):
    grid = (N // _BLK,)
    row = lambda shape: pl.BlockSpec(shape, lambda i: (i, 0))
    full = lambda a: pl.BlockSpec(a.shape, lambda i: tuple([0] * a.ndim))
    in_specs = [row((_BLK, 64)), pl.BlockSpec((2, _BLK, 32), lambda i: (0, i, 0)),
                row((_BLK, 1))] + [full(w) for w in weights]
    out_shape = [jax.ShapeDtypeStruct((N, 64), jnp.float32)]
    out_specs = [row((_BLK, 64))]
    if emit:
        out_shape += [jax.ShapeDtypeStruct((2, N, 32), jnp.float32)] * 2
        out_specs += [pl.BlockSpec((2, _BLK, 32), lambda i: (0, i, 0))] * 2
    return pl.pallas_call(functools.partial(_update_body, emit), grid=grid,
                          in_specs=in_specs, out_specs=out_specs,
                          out_shape=out_shape)(h, S3, deg2, *weights)


def _update_final_body(h_i, S3, deg2, cell_i, gid2, t2, ge8,
                       We2a, We2b, be2, Wn1a, Wn1b, bn1, Wn2, bn2,
                       Wp1, bp1, Wp2, bp2,
                       Wz1h, Wz1cell, Wz1t, Wz1c, bz1, Wz2, bz2, F_o):
    f32 = jnp.float32
    h = h_i[...]
    agg = _mmh(S3[0], We2a[...]) + _mmh(S3[1], We2b[...]) + deg2[...] * be2[...]
    hn = h + _mmd(jnp.maximum(_mmd(h, Wn1a[...]) + _mmd(agg, Wn1b[...]) + bn1[...], 0.0), Wn2[...]) + bn2[...]
    t = t2[...]
    iot = lax.broadcasted_iota(jnp.int32, (1, 32), 1).astype(f32)
    freqs = jnp.exp(iot * (-math.log(200.0) / 32.0))
    x = t * freqs
    te = jnp.concatenate([jnp.cos(x), jnp.sin(x)], axis=1)
    cond = _mmd(jnp.maximum(_mmd(ge8[...], Wp1[...]) + bp1[...], 0.0), Wp2[...]) + bp2[...]
    Gn = _mmd(te, Wz1t[...]) + _mmd(cond, Wz1c[...])
    gid = gid2[...]
    ohg = (lax.broadcasted_iota(jnp.int32, (_BLK, 64), 1) == gid).astype(f32)
    u1 = _mmd(hn, Wz1h[...]) + _mmd(cell_i[...], Wz1cell[...]) + _mmh(ohg, Gn) + bz1[...]
    F_o[...] = _mmd(jnp.maximum(u1, 0.0), Wz2[...]) + bz2[...]


def _update_final(h, S3, deg2, cell, gid2, t2, ge8, weights, N):
    grid = (N // _BLK,)
    row = lambda shape: pl.BlockSpec(shape, lambda i: (i, 0))
    full = lambda a: pl.BlockSpec(a.shape, lambda i: tuple([0] * a.ndim))
    in_specs = [row((_BLK, 64)), pl.BlockSpec((2, _BLK, 32), lambda i: (0, i, 0)),
                row((_BLK, 1)), row((_BLK, 64)), row((_BLK, 1)),
                full(t2), full(ge8)] + [full(w) for w in weights]
    return pl.pallas_call(_update_final_body, grid=grid, in_specs=in_specs,
                          out_specs=row((_BLK, 1)),
                          out_shape=jax.ShapeDtypeStruct((N, 1), jnp.float32))(
        h, S3, deg2, cell, gid2, t2, ge8, *weights)


def _final_body(h_i, cell_i, gid2, t2, ge8,
                Wp1, bp1, Wp2, bp2,
                Wz1h, Wz1cell, Wz1t, Wz1c, bz1, Wz2, bz2, F_o):
    f32 = jnp.float32
    t = t2[...]
    iot = lax.broadcasted_iota(jnp.int32, (1, 32), 1).astype(f32)
    freqs = jnp.exp(iot * (-math.log(200.0) / 32.0))
    x = t * freqs
    te = jnp.concatenate([jnp.cos(x), jnp.sin(x)], axis=1)
    cond = _mmd(jnp.maximum(_mmd(ge8[...], Wp1[...]) + bp1[...], 0.0), Wp2[...]) + bp2[...]
    Gn = _mmd(te, Wz1t[...]) + _mmd(cond, Wz1c[...])
    gid = gid2[...]
    ohg = (lax.broadcasted_iota(jnp.int32, (_BLK, 64), 1) == gid).astype(f32)
    u1 = _mmd(h_i[...], Wz1h[...]) + _mmd(cell_i[...], Wz1cell[...]) + _mmh(ohg, Gn) + bz1[...]
    F_o[...] = _mmd(jnp.maximum(u1, 0.0), Wz2[...]) + bz2[...]


def _final(h, cell, gid2, t2, ge8, weights, N):
    grid = (N // _BLK,)
    row = lambda shape: pl.BlockSpec(shape, lambda i: (i, 0))
    full = lambda a: pl.BlockSpec(a.shape, lambda i: tuple([0] * a.ndim))
    in_specs = [row((_BLK, 64)), row((_BLK, 64)), row((_BLK, 1)),
                full(t2), full(ge8)] + [full(w) for w in weights]
    return pl.pallas_call(_final_body, grid=grid, in_specs=in_specs,
                          out_specs=row((_BLK, 1)),
                          out_shape=jax.ShapeDtypeStruct((N, 1), jnp.float32))(
        h, cell, gid2, t2, ge8, *weights)


def _make_edge_kernel(N, E, compute_deg):
    NS = 16
    EperS = E // NS
    C = 400
    NCH = EperS // C
    RperS = -(-(N // NS) // 8) * 8
    NP = NS * RperS
    assert EperS % C == 0

    mesh = plsc.VectorSubcoreMesh(core_axis_name="c", subcore_axis_name="s")

    out_type = [jax.ShapeDtypeStruct((2 * NP, 32), jnp.float32)]
    if compute_deg:
        out_type.append(jax.ShapeDtypeStruct((N,), jnp.float32))

    scratch_types = [
        pltpu.VMEM((C,), jnp.int32),
        pltpu.VMEM((C,), jnp.int32),
        pltpu.VMEM((C,), jnp.int32),
        pltpu.VMEM((C, 32), jnp.float32),
        pltpu.VMEM((C, 32), jnp.float32),
        pltpu.VMEM((C,), jnp.float32),
        pltpu.VMEM_SHARED((NP, 32), jnp.float32),
    ]
    if compute_deg:
        scratch_types.append(pltpu.VMEM_SHARED((N,), jnp.float32))
    scratch_types += [pltpu.SemaphoreType.DMA, pltpu.SemaphoreType.DMA]

    def body(A_h, B_h, src_h, dst_h, *refs):
        if compute_deg:
            (S_out, deg_out, src_v, dst_v, dstg_v, bufA, bufB, ones_v,
             S_sh, deg_sh, semA, semB) = refs
        else:
            (S_out, src_v, dst_v, dstg_v, bufA, bufB, ones_v,
             S_sh, semA, semB) = refs
        c = lax.axis_index("c")
        s = lax.axis_index("s")
        cN = c * N
        cNP = c * NP
        zeros16 = jnp.zeros((16,), jnp.float32)

        def zrow(i, _):
            bufA[i, pl.ds(0, 16)] = zeros16
            bufA[i, pl.ds(16, 16)] = zeros16
            return _
        lax.fori_loop(0, C, zrow, None)

        def zv(i, _):
            ones_v[pl.ds(i * 16, 16)] = zeros16
            return _
        lax.fori_loop(0, C // 16, zv, None)

        r0 = s * RperS
        n_full = RperS // C
        rem = RperS - n_full * C
        for j in range(n_full):
            pltpu.sync_copy(bufA, S_sh.at[pl.ds(r0 + j * C, C)])
        if rem:
            pltpu.sync_copy(bufA.at[pl.ds(0, rem)], S_sh.at[pl.ds(r0 + n_full * C, rem)])

        if compute_deg:
            DEG_PER = N // 10
            dfull, drem = DEG_PER // C, DEG_PER % C

            @pl.when(jnp.logical_and(c == 0, s < 10))
            def _():
                for j in range(dfull):
                    pltpu.sync_copy(ones_v, deg_sh.at[pl.ds(s * DEG_PER + j * C, C)])
                if drem:
                    pltpu.sync_copy(ones_v.at[pl.ds(0, drem)],
                                    deg_sh.at[pl.ds(s * DEG_PER + dfull * C, drem)])

            ones16 = jnp.full((16,), 1.0, jnp.float32)

            def ov(i, _):
                ones_v[pl.ds(i * 16, 16)] = ones16
                return _
            lax.fori_loop(0, C // 16, ov, None)

        plsc.subcore_barrier()

        ebase = s * EperS

        def bf16_round(v):
            u = lax.bitcast_convert_type(v, jnp.int32)
            bit = jnp.bitwise_and(lax.shift_right_logical(u, 16), 1)
            r = jnp.bitwise_and(u + 32767 + bit, jnp.int32(-65536))
            return lax.bitcast_convert_type(r, jnp.float32)

        def chunk(k, _):
            b0 = ebase + k * C
            pltpu.sync_copy(src_h.at[pl.ds(b0, C)], src_v)
            pltpu.sync_copy(dst_h.at[pl.ds(b0, C)], dst_v)

            @plsc.parallel_loop(0, C // 16, 1, unroll=4)
            def _(i):
                src_v[pl.ds(i * 16, 16)] = src_v[pl.ds(i * 16, 16)] + cN
                dstg_v[pl.ds(i * 16, 16)] = dst_v[pl.ds(i * 16, 16)] + cN

            cpA = pltpu.async_copy(A_h.at[src_v], bufA, semA)
            cpB = pltpu.async_copy(B_h.at[dstg_v], bufB, semB)
            cpA.wait()
            cpB.wait()

            @plsc.parallel_loop(0, C, 1, unroll=8)
            def _(i):
                a0 = bufA[i, pl.ds(0, 16)]
                b0_ = bufB[i, pl.ds(0, 16)]
                bufA[i, pl.ds(0, 16)] = bf16_round(jnp.maximum(a0 + b0_, 0.0))
                a1 = bufA[i, pl.ds(16, 16)]
                b1_ = bufB[i, pl.ds(16, 16)]
                bufA[i, pl.ds(16, 16)] = bf16_round(jnp.maximum(a1 + b1_, 0.0))

            pltpu.sync_copy(bufA, S_sh.at[dst_v], add=True)
            if compute_deg:
                @pl.when(c == 0)
                def _():
                    pltpu.sync_copy(ones_v, deg_sh.at[dst_v], add=True)
            return _
        lax.fori_loop(0, NCH, chunk, None)

        plsc.subcore_barrier()

        o0 = cNP + r0
        for j in range(n_full):
            pltpu.sync_copy(S_sh.at[pl.ds(r0 + j * C, C)], bufA)
            pltpu.sync_copy(bufA, S_out.at[pl.ds(o0 + j * C, C)])
        if rem:
            pltpu.sync_copy(S_sh.at[pl.ds(r0 + n_full * C, rem)], bufA.at[pl.ds(0, rem)])
            pltpu.sync_copy(bufA.at[pl.ds(0, rem)], S_out.at[pl.ds(o0 + n_full * C, rem)])

        if compute_deg:
            DEG_PER2 = N // 10
            dfull2, drem2 = DEG_PER2 // C, DEG_PER2 % C

            @pl.when(jnp.logical_and(c == 0, s < 10))
            def _():
                for j in range(dfull2):
                    pltpu.sync_copy(deg_sh.at[pl.ds(s * DEG_PER2 + j * C, C)], ones_v)
                    pltpu.sync_copy(ones_v, deg_out.at[pl.ds(s * DEG_PER2 + j * C, C)])
                if drem2:
                    pltpu.sync_copy(deg_sh.at[pl.ds(s * DEG_PER2 + dfull2 * C, drem2)],
                                    ones_v.at[pl.ds(0, drem2)])
                    pltpu.sync_copy(ones_v.at[pl.ds(0, drem2)],
                                    deg_out.at[pl.ds(s * DEG_PER2 + dfull2 * C, drem2)])

    return pl.kernel(body, mesh=mesh, out_type=out_type,
                     scratch_types=scratch_types,
                     compiler_params=pltpu.CompilerParams(
                         use_tc_tiling_on_sc=False))


def kernel(t, global_energy, features_0, energy_corrupted, layer, edge_index,
           graph_id, params):
    N = features_0.shape[0]
    E = edge_index.shape[1]
    f32 = jnp.float32

    (Wp1, bp1), (Wp2, bp2) = params['particle']
    (Wc1, bc1), (Wc2, bc2) = params['cell']
    (Wi1, bi1), (Wi2, bi2) = params['init']
    (Wz1, bz1), (Wz2, bz2) = params['noise']

    Wp1_8 = jnp.zeros((8, 64), f32).at[:4].set(Wp1)
    Wc1a = jnp.zeros((8, 64), f32).at[:5].set(Wc1[:5])
    Wc1b = Wc1[5:21]
    le48 = jnp.zeros((48, 16), f32).at[:45].set(params['layer_emb'])
    row2 = lambda b: b.reshape(1, -1)

    feat8 = jnp.concatenate(
        [energy_corrupted, features_0, jnp.zeros((N, 3), f32)], axis=1)
    layer2 = layer.reshape(N, 1).astype(jnp.int32)
    gid2 = graph_id.reshape(N, 1).astype(jnp.int32)
    t2 = t.reshape(-1, 1)
    ge8 = jnp.zeros((global_energy.shape[0], 8), f32).at[:, :4].set(global_energy)

    (We1_0, be1_0), (We2_0, be2_0) = params['mpnn'][0][0]
    (Wn1_0, bn1_0), (Wn2_0, bn2_0) = params['mpnn'][0][1]
    (We1_1, be1_1), (We2_1, be2_1) = params['mpnn'][1][0]
    (Wn1_1, bn1_1), (Wn2_1, bn2_1) = params['mpnn'][1][1]

    enc_w = [Wc1a, Wc1b, row2(bc1), Wc2, row2(bc2),
             Wp1_8, row2(bp1), Wp2, row2(bp2),
             Wi1[:64], Wi1[64:128], Wi1[128:192], row2(bi1), Wi2, row2(bi2),
             We1_0[:64], We1_0[64:], row2(be1_0)]
    cell, h, A3, B3 = _encode(feat8, layer2, gid2, t2, ge8, le48, enc_w, N)

    src = edge_index[0].astype(jnp.int32)
    dst = edge_index[1].astype(jnp.int32)

    NP = 16 * (-(-(N // 16) // 8) * 8)

    edge_k1 = _make_edge_kernel(N, E, compute_deg=True)
    S_flat, deg = edge_k1(A3.reshape(2 * N, 32), B3.reshape(2 * N, 32), src, dst)
    deg2 = deg.reshape(N, 1)

    bf = lambda w: w.astype(jnp.bfloat16).astype(f32)
    upd_w0 = [bf(We2_0[:32]), bf(We2_0[32:]), row2(be2_0),
              Wn1_0[:64], Wn1_0[64:], row2(bn1_0), Wn2_0, row2(bn2_0),
              We1_1[:64], We1_1[64:], row2(be1_1)]
    h, A3, B3 = _update(h, S_flat.reshape(2, NP, 32), deg2, upd_w0, True, N)

    edge_k2 = _make_edge_kernel(N, E, compute_deg=False)
    S_flat = edge_k2(A3.reshape(2 * N, 32), B3.reshape(2 * N, 32), src, dst)
    if isinstance(S_flat, (list, tuple)):
        S_flat = S_flat[0]

    uf_w = [bf(We2_1[:32]), bf(We2_1[32:]), row2(be2_1),
            Wn1_1[:64], Wn1_1[64:], row2(bn1_1), Wn2_1, row2(bn2_1),
            Wp1_8, row2(bp1), Wp2, row2(bp2),
            Wz1[:64], Wz1[64:128], Wz1[128:192], Wz1[192:256],
            row2(bz1), Wz2, row2(bz2)]
    return _update_final(h, S_flat.reshape(2, NP, 32), deg2, cell, gid2, t2,
                         ge8, uf_w, N)
